# baseline (speedup 1.0000x reference)
import functools

import jax
import jax.numpy as jnp
from jax import lax
from jax.experimental import pallas as pl
from jax.experimental.pallas import tpu as pltpu
from jax.experimental.pallas import tpu_sc as plsc

B = 1024
K = 8192
D = 256
KT = 1024
NK = K // KT


def _assign_body(dp_ref, c_ref, idx_out_ref, best_val, best_idx):
    k = pl.program_id(0)

    @pl.when(k == 0)
    def _init():
        best_val[...] = jnp.full(best_val.shape, -jnp.inf, best_val.dtype)
        best_idx[...] = jnp.zeros(best_idx.shape, best_idx.dtype)

    s = lax.dot_general(
        dp_ref[...], c_ref[...], (((1,), (1,)), ((), ())),
        preferred_element_type=jnp.float32)
    m = jnp.max(s, axis=1, keepdims=True)
    cols = lax.broadcasted_iota(jnp.int32, s.shape, 1)
    local = jnp.min(jnp.where(s == m, cols, jnp.int32(K)), axis=1,
                    keepdims=True)
    gidx = local + k * KT
    prev = best_val[...]
    better = m > prev
    best_val[...] = jnp.where(better, m, prev)
    best_idx[...] = jnp.where(better, gidx, best_idx[...])

    @pl.when(k == pl.num_programs(0) - 1)
    def _fin():
        idx_out_ref[...] = best_idx[...]


def _assign(dp_n, c_n):
    return pl.pallas_call(
        _assign_body,
        grid=(NK,),
        in_specs=[
            pl.BlockSpec((B, D), lambda k: (0, 0)),
            pl.BlockSpec((KT, D), lambda k: (k, 0)),
        ],
        out_specs=pl.BlockSpec((B, 1), lambda k: (0, 0)),
        out_shape=jax.ShapeDtypeStruct((B, 1), jnp.int32),
        scratch_shapes=[
            pltpu.VMEM((B, 1), jnp.float32),
            pltpu.VMEM((B, 1), jnp.int32),
        ],
    )(dp_n, c_n)


def _make_gather():
    info = plsc.get_sparse_core_info()
    nc, ns = info.num_cores, info.num_subcores
    nw = nc * ns
    b_per_w = B // nw
    mesh = plsc.VectorSubcoreMesh(core_axis_name="c", subcore_axis_name="s")

    @functools.partial(
        pl.kernel, mesh=mesh,
        out_type=jax.ShapeDtypeStruct((B, D), jnp.float32),
        scratch_types=[
            pltpu.VMEM((b_per_w,), jnp.int32),
            pltpu.VMEM((b_per_w, D), jnp.float32),
            pltpu.SemaphoreType.DMA,
        ],
    )
    def gather_rows(idx_hbm, table_hbm, out_hbm, idx_v, rows_v, sem):
        wid = lax.axis_index("s") * nc + lax.axis_index("c")
        base = wid * b_per_w
        pltpu.sync_copy(idx_hbm.at[pl.ds(base, b_per_w)], idx_v)
        pltpu.async_copy(table_hbm.at[idx_v], rows_v, sem).wait()
        pltpu.sync_copy(rows_v, out_hbm.at[pl.ds(base, b_per_w)])

    return gather_rows


_gather = _make_gather()


def _l2n(x):
    n = jnp.sqrt(jnp.sum(x * x, axis=-1, keepdims=True))
    return x / jnp.maximum(n, 1e-8)


def kernel(datapoints, input_ids, batch_cos_sim, centroid):
    dp_n = _l2n(jax.lax.stop_gradient(datapoints))
    c_n = _l2n(centroid)
    idx = _assign(dp_n, c_n).reshape(B)
    return _gather(idx, centroid)


# speedup vs baseline: 1.0963x; 1.0963x over previous
import functools

import jax
import jax.numpy as jnp
from jax import lax
from jax.experimental import pallas as pl
from jax.experimental.pallas import tpu as pltpu
from jax.experimental.pallas import tpu_sc as plsc

B = 1024
K = 8192
D = 256
KT = 1024
NK = K // KT


def _assign_body(dp_ref, c_ref, idx_out_ref, best_val, best_idx):
    k = pl.program_id(0)

    @pl.when(k == 0)
    def _init():
        best_val[...] = jnp.full(best_val.shape, -jnp.inf, best_val.dtype)
        best_idx[...] = jnp.zeros(best_idx.shape, best_idx.dtype)

    s = lax.dot_general(
        dp_ref[...], c_ref[...], (((1,), (1,)), ((), ())),
        preferred_element_type=jnp.float32)
    m = jnp.max(s, axis=1, keepdims=True)
    cols = lax.broadcasted_iota(jnp.int32, s.shape, 1)
    local = jnp.min(jnp.where(s == m, cols, jnp.int32(K)), axis=1,
                    keepdims=True)
    gidx = local + k * KT
    prev = best_val[...]
    better = m > prev
    best_val[...] = jnp.where(better, m, prev)
    best_idx[...] = jnp.where(better, gidx, best_idx[...])

    @pl.when(k == pl.num_programs(0) - 1)
    def _fin():
        idx_out_ref[...] = best_idx[...].reshape(B)


def _assign(dp_b, c_b):
    return pl.pallas_call(
        _assign_body,
        grid=(NK,),
        in_specs=[
            pl.BlockSpec((B, D), lambda k: (0, 0)),
            pl.BlockSpec((KT, D), lambda k: (k, 0)),
        ],
        out_specs=pl.BlockSpec((B,), lambda k: (0,)),
        out_shape=jax.ShapeDtypeStruct((B,), jnp.int32),
        scratch_shapes=[
            pltpu.VMEM((B, 1), jnp.float32),
            pltpu.VMEM((B, 1), jnp.int32),
        ],
    )(dp_b, c_b)


def _make_gather():
    info = plsc.get_sparse_core_info()
    nc, ns = info.num_cores, info.num_subcores
    nw = nc * ns
    b_per_w = B // nw
    mesh = plsc.VectorSubcoreMesh(core_axis_name="c", subcore_axis_name="s")

    @functools.partial(
        pl.kernel, mesh=mesh,
        out_type=jax.ShapeDtypeStruct((B, D), jnp.float32),
        scratch_types=[
            pltpu.VMEM((b_per_w,), jnp.int32),
            pltpu.VMEM((b_per_w, D), jnp.float32),
            pltpu.SemaphoreType.DMA,
        ],
    )
    def gather_rows(idx_hbm, table_hbm, out_hbm, idx_v, rows_v, sem):
        wid = lax.axis_index("s") * nc + lax.axis_index("c")
        base = wid * b_per_w
        pltpu.sync_copy(idx_hbm.at[pl.ds(base, b_per_w)], idx_v)
        pltpu.async_copy(table_hbm.at[idx_v], rows_v, sem).wait()
        pltpu.sync_copy(rows_v, out_hbm.at[pl.ds(base, b_per_w)])

    return gather_rows


_gather = _make_gather()


def _l2n_bf16(x):
    n = jnp.sqrt(jnp.sum(x * x, axis=-1, keepdims=True))
    return (x / jnp.maximum(n, 1e-8)).astype(jnp.bfloat16)


def kernel(datapoints, input_ids, batch_cos_sim, centroid):
    dp_b = _l2n_bf16(jax.lax.stop_gradient(datapoints))
    c_b = _l2n_bf16(centroid)
    idx = _assign(dp_b, c_b)
    return _gather(idx, centroid)


# speedup vs baseline: 1.1963x; 1.0912x over previous
import jax
import jax.numpy as jnp
from jax import lax
from jax.experimental import pallas as pl
from jax.experimental.pallas import tpu as pltpu

B = 1024
K = 8192
D = 256
KT = 1024
NK = K // KT


def _body(dp_ref, c_ref, mx_ref, out_ref, best_val, best_idx, out_acc):
    k = pl.program_id(0)

    @pl.when(k == 0)
    def _init():
        best_val[...] = jnp.full(best_val.shape, -jnp.inf, best_val.dtype)
        best_idx[...] = jnp.zeros(best_idx.shape, best_idx.dtype)

    s = lax.dot_general(
        dp_ref[...], c_ref[...], (((1,), (1,)), ((), ())),
        preferred_element_type=jnp.float32)
    m = jnp.max(s, axis=1, keepdims=True)
    cols = lax.broadcasted_iota(jnp.int32, s.shape, 1)
    local = jnp.min(jnp.where(s == m, cols, jnp.int32(K)), axis=1,
                    keepdims=True)
    gidx = local + k * KT
    prev = best_val[...]
    better = m > prev
    best_val[...] = jnp.where(better, m, prev)
    best_idx[...] = jnp.where(better, gidx, best_idx[...])

    oh = jnp.where(cols == local, jnp.float32(1),
                   jnp.float32(0)).astype(jnp.bfloat16)
    cand = lax.dot_general(
        oh, c_ref[...], (((1,), (0,)), ((), ())),
        preferred_element_type=jnp.float32)
    nr = lax.dot_general(
        oh, mx_ref[...].astype(jnp.bfloat16), (((1,), (0,)), ((), ())),
        preferred_element_type=jnp.float32)
    out_acc[...] = jnp.where(better, cand * nr, out_acc[...])

    @pl.when(k == pl.num_programs(0) - 1)
    def _fin():
        out_ref[...] = out_acc[...]


def _assign_gather(dp_b, c_b, mx):
    return pl.pallas_call(
        _body,
        grid=(NK,),
        in_specs=[
            pl.BlockSpec((B, D), lambda k: (0, 0)),
            pl.BlockSpec((KT, D), lambda k: (k, 0)),
            pl.BlockSpec((KT, 1), lambda k: (k, 0)),
        ],
        out_specs=pl.BlockSpec((B, D), lambda k: (0, 0)),
        out_shape=jax.ShapeDtypeStruct((B, D), jnp.float32),
        scratch_shapes=[
            pltpu.VMEM((B, 1), jnp.float32),
            pltpu.VMEM((B, 1), jnp.int32),
            pltpu.VMEM((B, D), jnp.float32),
        ],
    )(dp_b, c_b, mx)


def kernel(datapoints, input_ids, batch_cos_sim, centroid):
    dp = jax.lax.stop_gradient(datapoints)
    ndp = jnp.maximum(jnp.sqrt(jnp.sum(dp * dp, axis=-1, keepdims=True)),
                      1e-8)
    dp_b = (dp / ndp).astype(jnp.bfloat16)
    mx = jnp.maximum(
        jnp.sqrt(jnp.sum(centroid * centroid, axis=-1, keepdims=True)), 1e-8)
    c_b = (centroid / mx).astype(jnp.bfloat16)
    return _assign_gather(dp_b, c_b, mx)


# speedup vs baseline: 1.2193x; 1.0192x over previous
import jax
import jax.numpy as jnp
from jax import lax
from jax.experimental import pallas as pl
from jax.experimental.pallas import tpu as pltpu

B = 1024
K = 8192
D = 256
KT = 1024
NK = K // KT


def _body(dp_ref, c_ref, cr_ref, out_ref, best_val, best_idx, out_acc):
    k = pl.program_id(0)

    @pl.when(k == 0)
    def _init():
        best_val[...] = jnp.full(best_val.shape, -jnp.inf, best_val.dtype)
        best_idx[...] = jnp.zeros(best_idx.shape, best_idx.dtype)

    s = lax.dot_general(
        dp_ref[...], c_ref[...], (((1,), (1,)), ((), ())),
        preferred_element_type=jnp.float32)
    m = jnp.max(s, axis=1, keepdims=True)
    cols = lax.broadcasted_iota(jnp.int32, s.shape, 1)
    local = jnp.min(jnp.where(s == m, cols, jnp.int32(K)), axis=1,
                    keepdims=True)
    gidx = local + k * KT
    prev = best_val[...]
    better = m > prev
    best_val[...] = jnp.where(better, m, prev)
    best_idx[...] = jnp.where(better, gidx, best_idx[...])

    oh = jnp.where(cols == local, jnp.float32(1),
                   jnp.float32(0)).astype(jnp.bfloat16)
    cand = lax.dot_general(
        oh, cr_ref[...], (((1,), (0,)), ((), ())),
        preferred_element_type=jnp.float32)
    out_acc[...] = jnp.where(better, cand, out_acc[...])

    @pl.when(k == pl.num_programs(0) - 1)
    def _fin():
        out_ref[...] = out_acc[...]


def _assign_gather(dp_b, c_b, c_r):
    return pl.pallas_call(
        _body,
        grid=(NK,),
        in_specs=[
            pl.BlockSpec((B, D), lambda k: (0, 0)),
            pl.BlockSpec((KT, D), lambda k: (k, 0)),
            pl.BlockSpec((KT, D), lambda k: (k, 0)),
        ],
        out_specs=pl.BlockSpec((B, D), lambda k: (0, 0)),
        out_shape=jax.ShapeDtypeStruct((B, D), jnp.float32),
        scratch_shapes=[
            pltpu.VMEM((B, 1), jnp.float32),
            pltpu.VMEM((B, 1), jnp.int32),
            pltpu.VMEM((B, D), jnp.float32),
        ],
    )(dp_b, c_b, c_r)


def kernel(datapoints, input_ids, batch_cos_sim, centroid):
    dp = jax.lax.stop_gradient(datapoints)
    ndp = jnp.maximum(jnp.sqrt(jnp.sum(dp * dp, axis=-1, keepdims=True)),
                      1e-8)
    dp_b = (dp / ndp).astype(jnp.bfloat16)
    mx = jnp.maximum(
        jnp.sqrt(jnp.sum(centroid * centroid, axis=-1, keepdims=True)), 1e-8)
    c_b = (centroid / mx).astype(jnp.bfloat16)
    c_r = centroid.astype(jnp.bfloat16)
    return _assign_gather(dp_b, c_b, c_r)


# speedup vs baseline: 1.3754x; 1.1280x over previous
import jax
import jax.numpy as jnp
from jax import lax
from jax.experimental import pallas as pl
from jax.experimental.pallas import tpu as pltpu

B = 1024
K = 8192
D = 256
KT = 1024
NK = K // KT
NJ = NK // 2 + 1


def _vpu_phase(s, t, valid, cr_ref, best_val, best_idx, out_acc):
    m = jnp.max(s, axis=1, keepdims=True)
    cols = lax.broadcasted_iota(jnp.int32, s.shape, 1)
    local = jnp.min(jnp.where(s == m, cols, jnp.int32(K)), axis=1,
                    keepdims=True)
    prev = best_val[...]
    better = jnp.logical_and(m > prev, valid)
    best_val[...] = jnp.where(better, m, prev)
    best_idx[...] = jnp.where(better, local + t * KT, best_idx[...])
    oh = jnp.where(cols == local, jnp.float32(1),
                   jnp.float32(0)).astype(jnp.bfloat16)
    cand = lax.dot_general(
        oh, cr_ref[...], (((1,), (0,)), ((), ())),
        preferred_element_type=jnp.float32)
    out_acc[...] = jnp.where(better, cand, out_acc[...])


def _body(dp_ref, cba_ref, cbb_ref, crb_ref, cra_ref, out_ref,
          best_val, best_idx, out_acc, s_a, s_b):
    j = pl.program_id(0)

    @pl.when(j == 0)
    def _init():
        best_val[...] = jnp.full(best_val.shape, -jnp.inf, best_val.dtype)
        best_idx[...] = jnp.zeros(best_idx.shape, best_idx.dtype)

    dp = dp_ref[...]
    s_a[...] = lax.dot_general(
        dp, cba_ref[...], (((1,), (1,)), ((), ())),
        preferred_element_type=jnp.float32)
    _vpu_phase(s_b[...], 2 * j - 1, j > 0, crb_ref,
               best_val, best_idx, out_acc)
    s_b[...] = lax.dot_general(
        dp, cbb_ref[...], (((1,), (1,)), ((), ())),
        preferred_element_type=jnp.float32)
    _vpu_phase(s_a[...], 2 * j, 2 * j <= NK - 1, cra_ref,
               best_val, best_idx, out_acc)

    @pl.when(j == NJ - 1)
    def _fin():
        out_ref[...] = out_acc[...]


def _assign_gather(dp_b, c_b, c_r):
    last = NK - 1
    return pl.pallas_call(
        _body,
        grid=(NJ,),
        in_specs=[
            pl.BlockSpec((B, D), lambda j: (0, 0)),
            pl.BlockSpec((KT, D), lambda j: (jnp.minimum(2 * j, last), 0)),
            pl.BlockSpec((KT, D),
                         lambda j: (jnp.minimum(2 * j + 1, last), 0)),
            pl.BlockSpec((KT, D),
                         lambda j: (jnp.maximum(2 * j - 1, 0), 0)),
            pl.BlockSpec((KT, D), lambda j: (jnp.minimum(2 * j, last), 0)),
        ],
        out_specs=pl.BlockSpec((B, D), lambda j: (0, 0)),
        out_shape=jax.ShapeDtypeStruct((B, D), jnp.float32),
        scratch_shapes=[
            pltpu.VMEM((B, 1), jnp.float32),
            pltpu.VMEM((B, 1), jnp.int32),
            pltpu.VMEM((B, D), jnp.float32),
            pltpu.VMEM((B, KT), jnp.float32),
            pltpu.VMEM((B, KT), jnp.float32),
        ],
    )(dp_b, c_b, c_b, c_r, c_r)


def kernel(datapoints, input_ids, batch_cos_sim, centroid):
    dp = jax.lax.stop_gradient(datapoints)
    ndp = jnp.maximum(jnp.sqrt(jnp.sum(dp * dp, axis=-1, keepdims=True)),
                      1e-8)
    dp_b = (dp / ndp).astype(jnp.bfloat16)
    mx = jnp.maximum(
        jnp.sqrt(jnp.sum(centroid * centroid, axis=-1, keepdims=True)), 1e-8)
    c_b = (centroid / mx).astype(jnp.bfloat16)
    c_r = centroid.astype(jnp.bfloat16)
    return _assign_gather(dp_b, c_b, c_r)
